# Initial kernel scaffold; baseline (speedup 1.0000x reference)
import functools

import jax
import jax.numpy as jnp
from jax import lax
from jax.experimental import pallas as pl
from jax.experimental.pallas import tpu as pltpu
from jax.experimental.pallas import tpu_sc as plsc

F32 = jnp.float32
NC, NS = 2, 16
NW = NC * NS


def _vmesh():
    return plsc.VectorSubcoreMesh(core_axis_name="c", subcore_axis_name="s")


def _sc_gather(table, idx):
    _, d = table.shape
    (b,) = idx.shape
    per_w = b // NW

    @functools.partial(
        pl.kernel,
        out_type=jax.ShapeDtypeStruct((b, d), F32),
        mesh=_vmesh(),
        scratch_types=[
            pltpu.VMEM((per_w,), jnp.int32),
            pltpu.VMEM((per_w, d), F32),
            pltpu.SemaphoreType.DMA,
        ],
    )
    def k(table_hbm, idx_hbm, out_hbm, idx_v, rows_v, sem):
        wid = lax.axis_index("s") * NC + lax.axis_index("c")
        base = wid * per_w
        pltpu.sync_copy(idx_hbm.at[pl.ds(base, per_w)], idx_v)
        pltpu.async_copy(table_hbm.at[idx_v], rows_v, sem).wait()
        pltpu.sync_copy(rows_v, out_hbm.at[pl.ds(base, per_w)])

    return k(table, idx)


def _sc_scatter(msg, dst, n_pad, n_chunks):
    e, co8 = msg.shape
    per_w = e // NW
    ch = per_w // n_chunks
    r = n_pad // NS
    zeros = jnp.zeros((n_pad, co8), F32)

    @functools.partial(
        pl.kernel,
        out_type=jax.ShapeDtypeStruct((NC, n_pad, co8), F32),
        mesh=_vmesh(),
        scratch_types=[
            pltpu.VMEM((ch,), jnp.int32),
            pltpu.VMEM((ch, co8), F32),
            pltpu.VMEM_SHARED((n_pad, co8), F32),
        ],
    )
    def k(msg_hbm, dst_hbm, z_hbm, out_hbm, idx_v, upd_v, acc_s):
        c = lax.axis_index("c")
        s = lax.axis_index("s")
        pltpu.sync_copy(z_hbm.at[pl.ds(s * r, r)], acc_s.at[pl.ds(s * r, r)])
        plsc.subcore_barrier()
        for j in range(n_chunks):
            base = (c * NS + s) * per_w + j * ch
            pltpu.sync_copy(dst_hbm.at[pl.ds(base, ch)], idx_v)
            pltpu.sync_copy(msg_hbm.at[pl.ds(base, ch)], upd_v)
            pltpu.sync_copy(upd_v, acc_s.at[idx_v], add=True)
        plsc.subcore_barrier()
        pltpu.sync_copy(acc_s.at[pl.ds(s * r, r)], out_hbm.at[c, pl.ds(s * r, r)])

    return k(msg, dst, zeros)


def _sc_pool(table, hex_t):
    _, co = table.shape
    _, lp = hex_t.shape
    per_w = lp // NW
    nv = co // 16

    @functools.partial(
        pl.kernel,
        out_type=jax.ShapeDtypeStruct((lp, co), F32),
        mesh=_vmesh(),
        scratch_types=[
            pltpu.VMEM((per_w,), jnp.int32),
            pltpu.VMEM((per_w, co), F32),
            pltpu.VMEM((per_w, co), F32),
            pltpu.SemaphoreType.DMA,
        ],
    )
    def k(tab_hbm, hex_hbm, out_hbm, idx_v, acc_v, buf_v, sem):
        wid = lax.axis_index("s") * NC + lax.axis_index("c")
        base = wid * per_w
        pltpu.sync_copy(hex_hbm.at[0, pl.ds(base, per_w)], idx_v)
        pltpu.async_copy(tab_hbm.at[idx_v], acc_v, sem).wait()
        for j in range(1, 7):
            pltpu.sync_copy(hex_hbm.at[j, pl.ds(base, per_w)], idx_v)
            pltpu.async_copy(tab_hbm.at[idx_v], buf_v, sem).wait()

            def body(row, _):
                for q in range(nv):
                    sl = pl.ds(q * 16, 16)
                    acc_v[row, sl] = jnp.maximum(acc_v[row, sl], buf_v[row, sl])
                return 0

            lax.fori_loop(0, per_w, body, 0)
        pltpu.sync_copy(acc_v, out_hbm.at[pl.ds(base, per_w)])

    return k(table, hex_t)


def _tc_msg(xs, pseudo, g, mu_t, sg_t):
    e, ci = xs.shape
    co3 = g.shape[1]
    co = co3 // 3
    co8 = co + 8
    blk = 1024 if e % 1024 == 0 else 768
    grid = e // blk

    def body(xs_ref, ps_ref, g_ref, mu_ref, sg_ref, out_ref):
        xj = jnp.dot(xs_ref[...], g_ref[...], preferred_element_type=F32)
        ps = ps_ref[...]
        mu = mu_ref[...]
        inv = 1.0 / (sg_ref[...] ** 2 + 1e-15)
        lin = -2.0 * mu * inv
        const = jnp.sum(mu * mu * inv, axis=0, keepdims=True)
        dist = (jnp.dot(ps * ps, inv, preferred_element_type=F32)
                + jnp.dot(ps, lin, preferred_element_type=F32) + const)
        w = jnp.exp(-0.5 * dist)
        msg = (w[:, 0:1] * xj[:, 0:co]
               + w[:, 1:2] * xj[:, co:2 * co]
               + w[:, 2:3] * xj[:, 2 * co:3 * co])
        out_ref[...] = jnp.concatenate(
            [msg, jnp.ones((blk, 1), F32), jnp.zeros((blk, 7), F32)], axis=1)

    return pl.pallas_call(
        body,
        grid=(grid,),
        in_specs=[
            pl.BlockSpec((blk, ci), lambda i: (i, 0)),
            pl.BlockSpec((blk, 3), lambda i: (i, 0)),
            pl.BlockSpec((ci, co3), lambda i: (0, 0)),
            pl.BlockSpec((3, 3), lambda i: (0, 0)),
            pl.BlockSpec((3, 3), lambda i: (0, 0)),
        ],
        out_specs=pl.BlockSpec((blk, co8), lambda i: (i, 0)),
        out_shape=jax.ShapeDtypeStruct((e, co8), F32),
    )(xs, pseudo, g, mu_t, sg_t)


def _tc_combine(s_part, xin, root, bias, n_pad):
    _, _, co8 = s_part.shape
    co = co8 - 8
    ci = xin.shape[1]
    blk = 1024
    grid = n_pad // blk

    def body(sp_ref, x_ref, root_ref, b_ref, out_ref):
        sp = sp_ref[...]
        ssum = sp[0] + sp[1]
        cnt = ssum[:, co:co + 1]
        agg = ssum[:, 0:co] / jnp.maximum(cnt, 1.0)
        rt = jnp.dot(x_ref[...], root_ref[...], preferred_element_type=F32)
        out_ref[...] = jnp.maximum(agg + rt + b_ref[...], 0.0)

    return pl.pallas_call(
        body,
        grid=(grid,),
        in_specs=[
            pl.BlockSpec((2, blk, co8), lambda i: (0, i, 0)),
            pl.BlockSpec((blk, ci), lambda i: (i, 0)),
            pl.BlockSpec((ci, co), lambda i: (0, 0)),
            pl.BlockSpec((1, co), lambda i: (0, 0)),
        ],
        out_specs=pl.BlockSpec((blk, co), lambda i: (i, 0)),
        out_shape=jax.ShapeDtypeStruct((n_pad, co), F32),
    )(s_part, xin, root, bias.reshape(1, co))


def _tc_head(h, fc_w, fc_b, fc2_w, fc2_b, n_valid):
    rows, _ = h.shape

    def body(h_ref, w1_ref, b1_ref, w2_ref, b2_ref, out_ref):
        hv = h_ref[...]
        ridx = lax.broadcasted_iota(jnp.int32, (rows, 1), 0)
        valid = ridx < n_valid
        hmax = jnp.max(jnp.where(valid, hv, -jnp.inf), axis=0, keepdims=True)
        hmean = jnp.sum(jnp.where(valid, hv, 0.0), axis=0,
                        keepdims=True) / float(n_valid)
        xc = jnp.concatenate([hmax, hmean], axis=1)
        o = jnp.maximum(
            jnp.dot(xc, w1_ref[...], preferred_element_type=F32) + b1_ref[...],
            0.0)
        out_ref[...] = (jnp.dot(o, w2_ref[...], preferred_element_type=F32)
                        + b2_ref[...])

    return pl.pallas_call(
        body,
        out_shape=jax.ShapeDtypeStruct((1, 1), F32),
    )(h, fc_w, fc_b.reshape(1, -1), fc2_w, fc2_b.reshape(1, 1))


_LEVELS = (
    (40962, 245760, 10242, 4, 41984, 10496),
    (10242, 61440, 2562, 2, 11264, 2816),
    (2562, 15360, 642, 1, 3072, 768),
    (642, 3840, 162, 1, 1024, 256),
)


def kernel(x, edge_index, batch, edge_index_5, edge_index_4, edge_index_3,
           pseudo_6, pseudo_5, pseudo_4, pseudo_3, hex_6, hex_5, hex_4, hex_3,
           g1, mu1, sigma1, root1, b1, g2, mu2, sigma2, root2, b2,
           g3, mu3, sigma3, root3, b3, g4, mu4, sigma4, root4, b4,
           fcW, fcb, fc2W, fc2b):
    params = (
        (edge_index, pseudo_6, hex_6, g1, mu1, sigma1, root1, b1),
        (edge_index_5, pseudo_5, hex_5, g2, mu2, sigma2, root2, b2),
        (edge_index_4, pseudo_4, hex_4, g3, mu3, sigma3, root3, b3),
        (edge_index_3, pseudo_3, hex_3, g4, mu4, sigma4, root4, b4),
    )
    h = x
    for lvl, (ei, ps, hx, g, mu, sg, root, bb) in zip(_LEVELS, params):
        _, _, l_out, n_chunks, n_pad, l_pad = lvl
        src = ei[0]
        dst = ei[1]
        xs = _sc_gather(h, src)
        msg = _tc_msg(xs, ps, g, mu.T, sg.T)
        s_part = _sc_scatter(msg, dst, n_pad, n_chunks)
        out = _tc_combine(s_part, h, root, bb, n_pad)
        hex_t = jnp.pad(hx[:l_out].T, ((0, 0), (0, l_pad - l_out)))
        h = _sc_pool(out, hex_t)
    o = _tc_head(h, fcW, fcb, fc2W, fc2b, 162)
    return o.reshape(1)


# speedup vs baseline: 2.2965x; 2.2965x over previous
import functools

import jax
import jax.numpy as jnp
from jax import lax
from jax.experimental import pallas as pl
from jax.experimental.pallas import tpu as pltpu
from jax.experimental.pallas import tpu_sc as plsc

F32 = jnp.float32
NC, NS = 2, 16
NW = NC * NS


def _vmesh():
    return plsc.VectorSubcoreMesh(core_axis_name="c", subcore_axis_name="s")


def _gate(a, dep):
    z = lax.optimization_barrier(dep.reshape(-1)[0:1] * 0)[0]
    return a + z.astype(a.dtype)


def _sc_gather(table, idx):
    _, d = table.shape
    (b,) = idx.shape
    per_w = b // NW

    @functools.partial(
        pl.kernel,
        out_type=jax.ShapeDtypeStruct((b, d), F32),
        mesh=_vmesh(),
        compiler_params=pltpu.CompilerParams(use_tc_tiling_on_sc=False),
        scratch_types=[
            pltpu.VMEM((per_w,), jnp.int32),
            pltpu.VMEM((per_w, d), F32),
            pltpu.SemaphoreType.DMA,
        ],
    )
    def k(table_hbm, idx_hbm, out_hbm, idx_v, rows_v, sem):
        wid = lax.axis_index("s") * NC + lax.axis_index("c")
        base = wid * per_w
        pltpu.sync_copy(idx_hbm.at[pl.ds(base, per_w)], idx_v)
        pltpu.async_copy(table_hbm.at[idx_v], rows_v, sem).wait()
        pltpu.sync_copy(rows_v, out_hbm.at[pl.ds(base, per_w)])

    return k(table, idx)


def _sc_scatter(msg, dst2d, n_pad, n_chunks):
    e, co8 = msg.shape
    rows_all = dst2d.shape[0]
    per_w_r = rows_all // NW
    ch_r = per_w_r // n_chunks
    ch = ch_r * 128
    r = n_pad // NS
    zeros = _gate(jnp.zeros((n_pad, co8), F32), msg)

    @functools.partial(
        pl.kernel,
        out_type=jax.ShapeDtypeStruct((NC, n_pad, co8), F32),
        mesh=_vmesh(),
        compiler_params=pltpu.CompilerParams(use_tc_tiling_on_sc=False),
        scratch_types=[
            pltpu.VMEM((ch_r, 128), jnp.int32),
            pltpu.VMEM((ch, co8), F32),
            pltpu.VMEM_SHARED((n_pad, co8), F32),
        ],
    )
    def k(msg_hbm, dst_hbm, z_hbm, out_hbm, idx_v, upd_v, acc_s):
        c = lax.axis_index("c")
        s = lax.axis_index("s")
        pltpu.sync_copy(z_hbm.at[pl.ds(s * r, r)], acc_s.at[pl.ds(s * r, r)])
        plsc.subcore_barrier()
        for j in range(n_chunks):
            rbase = (c * NS + s) * per_w_r + j * ch_r
            pltpu.sync_copy(dst_hbm.at[pl.ds(rbase, ch_r)], idx_v)
            pltpu.sync_copy(msg_hbm.at[pl.ds(rbase * 128, ch)], upd_v)

            def scat(j2, _):
                pltpu.sync_copy(upd_v.at[pl.ds(j2 * 128, 128)],
                                acc_s.at[idx_v.at[j2]], add=True)
                return 0

            lax.fori_loop(0, ch_r, scat, 0)
        plsc.subcore_barrier()
        pltpu.sync_copy(acc_s.at[pl.ds(s * r, r)], out_hbm.at[c, pl.ds(s * r, r)])

    return k(msg, dst2d, zeros)


def _sc_scatter_nodesplit(msg, idx2_all, half, acc_r, n_out, n_chunks):
    e, co8 = msg.shape
    rows_all = idx2_all.shape[1]
    per_t_r = rows_all // NS
    ch_r = per_t_r // n_chunks
    ch = ch_r * 128
    r = acc_r // NS
    zeros = _gate(jnp.zeros((acc_r, co8), F32), msg)

    @functools.partial(
        pl.kernel,
        out_type=jax.ShapeDtypeStruct((n_out, co8), F32),
        mesh=_vmesh(),
        compiler_params=pltpu.CompilerParams(use_tc_tiling_on_sc=False),
        scratch_types=[
            pltpu.VMEM((ch_r, 128), jnp.int32),
            pltpu.VMEM((ch, co8), F32),
            pltpu.VMEM_SHARED((acc_r, co8), F32),
        ],
    )
    def k(msg_hbm, idx_hbm, z_hbm, out_hbm, idx_v, upd_v, acc_s):
        c = lax.axis_index("c")
        s = lax.axis_index("s")
        pltpu.sync_copy(z_hbm.at[pl.ds(s * r, r)], acc_s.at[pl.ds(s * r, r)])
        plsc.subcore_barrier()
        for j in range(n_chunks):
            rbase = s * per_t_r + j * ch_r
            pltpu.sync_copy(idx_hbm.at[c, pl.ds(rbase, ch_r)], idx_v)
            pltpu.sync_copy(msg_hbm.at[pl.ds(rbase * 128, ch)], upd_v)

            def scat(j2, _):
                pltpu.sync_copy(upd_v.at[pl.ds(j2 * 128, 128)],
                                acc_s.at[idx_v.at[j2]], add=True)
                return 0

            lax.fori_loop(0, ch_r, scat, 0)
        plsc.subcore_barrier()
        rw = half // NS
        pltpu.sync_copy(acc_s.at[pl.ds(s * rw, rw)],
                        out_hbm.at[pl.ds(c * half + s * rw, rw)])

    return k(msg, idx2_all, zeros)


def _sc_count_nodesplit(idx2_all, gate, half, acc_r, n_out, n_chunks):
    rows_all = idx2_all.shape[1]
    per_t_r = rows_all // NS
    ch_r = per_t_r // n_chunks
    r = acc_r // NS
    zeros = _gate(jnp.zeros((acc_r, 8), F32), gate)
    ones = _gate(jnp.ones((128, 8), F32), gate)

    @functools.partial(
        pl.kernel,
        out_type=jax.ShapeDtypeStruct((n_out, 8), F32),
        mesh=_vmesh(),
        compiler_params=pltpu.CompilerParams(use_tc_tiling_on_sc=False),
        scratch_types=[
            pltpu.VMEM((ch_r, 128), jnp.int32),
            pltpu.VMEM((128, 8), F32),
            pltpu.VMEM_SHARED((acc_r, 8), F32),
        ],
    )
    def k(idx_hbm, gate_hbm, z_hbm, ones_hbm, out_hbm, idx_v, ones_v, acc_s):
        c = lax.axis_index("c")
        s = lax.axis_index("s")
        pltpu.sync_copy(z_hbm.at[pl.ds(s * r, r)], acc_s.at[pl.ds(s * r, r)])
        pltpu.sync_copy(ones_hbm, ones_v)
        plsc.subcore_barrier()
        for j in range(n_chunks):
            rbase = s * per_t_r + j * ch_r
            pltpu.sync_copy(idx_hbm.at[c, pl.ds(rbase, ch_r)], idx_v)

            def scat(j2, _):
                pltpu.sync_copy(ones_v, acc_s.at[idx_v.at[j2]], add=True)
                return 0

            lax.fori_loop(0, ch_r, scat, 0)
        plsc.subcore_barrier()
        rw = half // NS
        pltpu.sync_copy(acc_s.at[pl.ds(s * rw, rw)],
                        out_hbm.at[pl.ds(c * half + s * rw, rw)])

    return k(idx2_all, gate, zeros, ones)


def _sc_pool(table, hex_t):
    _, co = table.shape
    _, lp = hex_t.shape
    per_w = lp // NW
    nv = co // 16

    @functools.partial(
        pl.kernel,
        out_type=jax.ShapeDtypeStruct((lp, co), F32),
        mesh=_vmesh(),
        compiler_params=pltpu.CompilerParams(use_tc_tiling_on_sc=False),
        scratch_types=[
            pltpu.VMEM((per_w,), jnp.int32),
            pltpu.VMEM((per_w, co), F32),
            pltpu.VMEM((per_w, co), F32),
            pltpu.SemaphoreType.DMA,
        ],
    )
    def k(tab_hbm, hex_hbm, out_hbm, idx_v, acc_v, buf_v, sem):
        wid = lax.axis_index("s") * NC + lax.axis_index("c")
        base = wid * per_w
        pltpu.sync_copy(hex_hbm.at[0, pl.ds(base, per_w)], idx_v)
        pltpu.async_copy(tab_hbm.at[idx_v], acc_v, sem).wait()
        for j in range(1, 7):
            pltpu.sync_copy(hex_hbm.at[j, pl.ds(base, per_w)], idx_v)
            pltpu.async_copy(tab_hbm.at[idx_v], buf_v, sem).wait()

            def body(row, _):
                for q in range(nv):
                    sl = pl.ds(q * 16, 16)
                    acc_v[row, sl] = jnp.maximum(acc_v[row, sl], buf_v[row, sl])
                return 0

            lax.fori_loop(0, per_w, body, 0)
        pltpu.sync_copy(acc_v, out_hbm.at[pl.ds(base, per_w)])

    return k(table, hex_t)


def _tc_remap(dst2d, half):
    rows = dst2d.shape[0]
    blk_r = 192 if rows % 192 == 0 else rows
    grid = rows // blk_r

    def body(d_ref, out_ref):
        d = d_ref[...]
        lane = lax.broadcasted_iota(jnp.int32, (blk_r, 128), 1)
        for c in range(NC):
            r = d - c * half
            ok = (r >= 0) & (r < half)
            out_ref[c] = jnp.where(ok, r, half + lane)

    return pl.pallas_call(
        body,
        grid=(grid,),
        in_specs=[pl.BlockSpec((blk_r, 128), lambda i: (i, 0))],
        out_specs=pl.BlockSpec((NC, blk_r, 128), lambda i: (0, i, 0)),
        out_shape=jax.ShapeDtypeStruct((NC, rows, 128), jnp.int32),
    )(dst2d)


def _tc_msg(xs, pseudo, g, mu_t, sg_t, e_real, count_col=True):
    e, ci = xs.shape
    co3 = g.shape[1]
    co = co3 // 3
    co8 = co + 8 if count_col else co
    blk = 1024 if e % 1024 == 0 else 768
    grid = e // blk

    def body(xs_ref, ps_ref, g_ref, mu_ref, sg_ref, out_ref):
        xj = jnp.dot(xs_ref[...], g_ref[...], preferred_element_type=F32, precision=lax.Precision.HIGHEST)
        ps = ps_ref[...]
        mu = mu_ref[...]
        inv = 1.0 / (sg_ref[...] ** 2 + 1e-15)
        lin = -2.0 * mu * inv
        const = jnp.sum(mu * mu * inv, axis=0, keepdims=True)
        dist = (jnp.dot(ps * ps, inv, preferred_element_type=F32, precision=lax.Precision.HIGHEST)
                + jnp.dot(ps, lin, preferred_element_type=F32, precision=lax.Precision.HIGHEST) + const)
        w = jnp.exp(-0.5 * dist)
        msg = (w[:, 0:1] * xj[:, 0:co]
               + w[:, 1:2] * xj[:, co:2 * co]
               + w[:, 2:3] * xj[:, 2 * co:3 * co])
        row = (pl.program_id(0) * blk
               + lax.broadcasted_iota(jnp.int32, (blk, 1), 0))
        valid = row < e_real
        if count_col:
            msg = jnp.concatenate(
                [msg, jnp.ones((blk, 1), F32), jnp.zeros((blk, 7), F32)],
                axis=1)
        out_ref[...] = jnp.where(valid, msg, 0.0)

    return pl.pallas_call(
        body,
        grid=(grid,),
        in_specs=[
            pl.BlockSpec((blk, ci), lambda i: (i, 0)),
            pl.BlockSpec((blk, 3), lambda i: (i, 0)),
            pl.BlockSpec((ci, co3), lambda i: (0, 0)),
            pl.BlockSpec((3, 3), lambda i: (0, 0)),
            pl.BlockSpec((3, 3), lambda i: (0, 0)),
        ],
        out_specs=pl.BlockSpec((blk, co8), lambda i: (i, 0)),
        out_shape=jax.ShapeDtypeStruct((e, co8), F32),
    )(xs, pseudo, g, mu_t, sg_t)


def _tc_combine(s_part, cnt, xin, root, bias, n_pad, blk):
    co8 = s_part.shape[-1]
    two_plane = s_part.ndim == 3
    co = co8 - 8 if cnt is None else co8
    ci = xin.shape[1]
    grid = n_pad // blk

    def tail(ssum, cntcol, x, root, b):
        agg = ssum[:, 0:co] / jnp.maximum(cntcol, 1.0)
        rt = jnp.dot(x, root, preferred_element_type=F32, precision=lax.Precision.HIGHEST)
        return jnp.maximum(agg + rt + b, 0.0)

    if cnt is None:
        def body(sp_ref, x_ref, root_ref, b_ref, out_ref):
            sp = sp_ref[...]
            ssum = sp[0] + sp[1] if two_plane else sp
            out_ref[...] = tail(ssum, ssum[:, co:co + 1], x_ref[...],
                                root_ref[...], b_ref[...])
        sp_spec = (pl.BlockSpec((2, blk, co8), lambda i: (0, i, 0))
                   if two_plane else pl.BlockSpec((blk, co8), lambda i: (i, 0)))
        ins = [s_part, xin, root, bias.reshape(1, co)]
        in_specs = [
            sp_spec,
            pl.BlockSpec((blk, ci), lambda i: (i, 0)),
            pl.BlockSpec((ci, co), lambda i: (0, 0)),
            pl.BlockSpec((1, co), lambda i: (0, 0)),
        ]
    else:
        def body(sp_ref, cnt_ref, x_ref, root_ref, b_ref, out_ref):
            out_ref[...] = tail(sp_ref[...], cnt_ref[...][:, 0:1], x_ref[...],
                                root_ref[...], b_ref[...])
        ins = [s_part, cnt, xin, root, bias.reshape(1, co)]
        in_specs = [
            pl.BlockSpec((blk, co8), lambda i: (i, 0)),
            pl.BlockSpec((blk, 8), lambda i: (i, 0)),
            pl.BlockSpec((blk, ci), lambda i: (i, 0)),
            pl.BlockSpec((ci, co), lambda i: (0, 0)),
            pl.BlockSpec((1, co), lambda i: (0, 0)),
        ]

    return pl.pallas_call(
        body,
        grid=(grid,),
        in_specs=in_specs,
        out_specs=pl.BlockSpec((blk, co), lambda i: (i, 0)),
        out_shape=jax.ShapeDtypeStruct((n_pad, co), F32),
    )(*ins)


def _tc_head(h, fc_w, fc_b, fc2_w, fc2_b, n_valid):
    rows, _ = h.shape

    def body(h_ref, w1_ref, b1_ref, w2_ref, b2_ref, out_ref):
        hv = h_ref[...]
        ridx = lax.broadcasted_iota(jnp.int32, (rows, 1), 0)
        valid = ridx < n_valid
        hmax = jnp.max(jnp.where(valid, hv, -jnp.inf), axis=0, keepdims=True)
        hmean = jnp.sum(jnp.where(valid, hv, 0.0), axis=0,
                        keepdims=True) / float(n_valid)
        xc = jnp.concatenate([hmax, hmean], axis=1)
        o = jnp.maximum(
            jnp.dot(xc, w1_ref[...], preferred_element_type=F32, precision=lax.Precision.HIGHEST) + b1_ref[...],
            0.0)
        out_ref[...] = (jnp.dot(o, w2_ref[...], preferred_element_type=F32, precision=lax.Precision.HIGHEST)
                        + b2_ref[...])

    return pl.pallas_call(
        body,
        out_shape=jax.ShapeDtypeStruct((1, 1), F32),
    )(h, fc_w, fc_b.reshape(1, -1), fc2_w, fc2_b.reshape(1, 1))


_LEVELS = (
    (40962, 245760, 245760, 10242, 8, 41984, 10496, 1024),
    (10242, 61440, 61440, 2562, 3, 10368, 2816, 648),
    (2562, 15360, 16384, 642, 1, 3072, 768, 1024),
    (642, 3840, 4096, 162, 1, 1024, 256, 1024),
)


def kernel(x, edge_index, batch, edge_index_5, edge_index_4, edge_index_3,
           pseudo_6, pseudo_5, pseudo_4, pseudo_3, hex_6, hex_5, hex_4, hex_3,
           g1, mu1, sigma1, root1, b1, g2, mu2, sigma2, root2, b2,
           g3, mu3, sigma3, root3, b3, g4, mu4, sigma4, root4, b4,
           fcW, fcb, fc2W, fc2b):
    params = (
        (edge_index, pseudo_6, hex_6, g1, mu1, sigma1, root1, b1),
        (edge_index_5, pseudo_5, hex_5, g2, mu2, sigma2, root2, b2),
        (edge_index_4, pseudo_4, hex_4, g3, mu3, sigma3, root3, b3),
        (edge_index_3, pseudo_3, hex_3, g4, mu4, sigma4, root4, b4),
    )
    h = x
    for li, (lvl, (ei, ps, hx, g, mu, sg, root, bb)) in enumerate(
            zip(_LEVELS, params)):
        _, e_real, e_pad, l_out, n_chunks, n_pad, l_pad, cblk = lvl
        src_idx = _gate(jnp.pad(ei[0], (0, e_pad - e_real)), h)
        xs = _sc_gather(h, src_idx)
        dst2d = _gate(jnp.pad(ei[1], (0, e_pad - e_real)), xs).reshape(-1, 128)
        if li == 0:
            msg = _tc_msg(xs, ps, g, mu.T, sg.T, e_real, count_col=False)
            idx2_all = _tc_remap(dst2d, 20608)
            s_part = _sc_scatter_nodesplit(
                msg, idx2_all, 20608, 20736, n_pad, n_chunks)
            cnt = _sc_count_nodesplit(idx2_all, s_part[:8, :8], 20608,
                                      20736, n_pad, n_chunks)
            out = _tc_combine(s_part, cnt, h, root, bb, n_pad, cblk)
        else:
            msg = _tc_msg(xs, ps, g, mu.T, sg.T, e_real)
            s_part = _sc_scatter(msg, dst2d, n_pad, n_chunks)
            out = _tc_combine(s_part, None, h, root, bb, n_pad, cblk)
        hex_t = _gate(jnp.pad(hx[:l_out].T, ((0, 0), (0, l_pad - l_out))), out)
        h = _sc_pool(out, hex_t)
    o = _tc_head(h, fcW, fcb, fc2W, fc2b, 162)
    return o.reshape(1)
